# baseline (speedup 1.0000x reference)
import functools

import numpy as np
import jax
import jax.numpy as jnp
from jax import lax
from jax.experimental import pallas as pl
from jax.experimental.pallas import tpu as pltpu
from jax.experimental.pallas import tpu_sc as plsc

NL = 10000
NC = 10000
E = 160000
D = 256
NQ = 4
H = D // NQ
ITERS = 3

NCORES = 2
NPASS = NQ // NCORES
NTILES = 16
CHUNK = 128
EP_TILE = 10240
NCHUNK = EP_TILE // CHUNK
E_PAD = EP_TILE * NTILES
ACC_ROWS = 10112
RPT = ACC_ROWS // NTILES


NBUF = 4


def _segsum_body(x_hbm, gidx_hbm, sidx_hbm, zeros_hbm, out_hbm,
                 gidx_v, sidx_v, rows_v, acc_s,
                 g0, g1, g2, g3, s0, s1, s2, s3):
    c = lax.axis_index("c")
    s = lax.axis_index("s")
    gsems = (g0, g1, g2, g3)
    ssems = (s0, s1, s2, s3)
    pltpu.sync_copy(sidx_hbm.at[s], sidx_v)

    for p in range(NPASS):
        q = c * NPASS + p
        pltpu.sync_copy(gidx_hbm.at[q, s], gidx_v)
        pltpu.sync_copy(zeros_hbm.at[pl.ds(s * RPT, RPT)],
                        acc_s.at[pl.ds(s * RPT, RPT)])
        plsc.subcore_barrier()

        for b in range(NBUF):
            pltpu.async_copy(x_hbm.at[gidx_v.at[b]], rows_v.at[b], gsems[b])

        def outer(o, carry):
            for b in range(NBUF):
                j = o * NBUF + b
                pltpu.make_async_copy(x_hbm.at[gidx_v.at[j]], rows_v.at[b],
                                      gsems[b]).wait()
                bp = (b - 1) % NBUF
                jp = j - 1 + NBUF

                @pl.when(jnp.logical_and(jp >= NBUF, jp < NCHUNK))
                def _():
                    pltpu.async_copy(x_hbm.at[gidx_v.at[jp]], rows_v.at[bp],
                                     gsems[bp])
            return carry

        lax.fori_loop(0, NCHUNK // NBUF, outer, 0)

        plsc.subcore_barrier()
        pltpu.sync_copy(acc_s.at[pl.ds(s * RPT, RPT)],
                        out_hbm.at[q, pl.ds(s * RPT, RPT)])


@functools.cache
def _make_segsum():
    return pl.kernel(
        _segsum_body,
        out_type=jax.ShapeDtypeStruct((NQ, ACC_ROWS, H), jnp.float32),
        mesh=plsc.VectorSubcoreMesh(core_axis_name="c", subcore_axis_name="s",
                                    num_cores=NCORES, num_subcores=NTILES),
        scratch_types=[
            pltpu.VMEM((NCHUNK, CHUNK), jnp.int32),
            pltpu.VMEM((NCHUNK, CHUNK), jnp.int32),
            pltpu.VMEM((NBUF, CHUNK, 2 * H), jnp.float32),
            pltpu.VMEM_SHARED((ACC_ROWS, H), jnp.float32),
        ] + [pltpu.SemaphoreType.DMA] * (2 * NBUF),
        compiler_params=pltpu.CompilerParams(use_tc_tiling_on_sc=False),
    )


def _segsum(*args):
    return _make_segsum()(*args)


def _layernorm(h, g, beta):
    mu = jnp.mean(h, axis=-1, keepdims=True)
    var = jnp.mean((h - mu) * (h - mu), axis=-1, keepdims=True)
    return (h - mu) * lax.rsqrt(var + 1e-5) * g + beta


def _cat(ref):
    return jnp.concatenate([ref[q] for q in range(NQ)], axis=-1)


def _mlp_c_body(eps_ref, x_ref, h_ref, w1_ref, b1_ref, w2_ref, b2_ref,
                g_ref, beta_ref, o_ref):
    eps1 = eps_ref[0]
    pre = eps1 * _cat(x_ref) + _cat(h_ref)
    a = jnp.dot(pre, w1_ref[...], preferred_element_type=jnp.float32) + b1_ref[...]
    a = jnp.maximum(a, 0.0)
    hb = jnp.dot(a, w2_ref[...], preferred_element_type=jnp.float32) + b2_ref[...]
    y = _layernorm(hb, g_ref[...], beta_ref[...])
    for q in range(NQ):
        o_ref[q] = y[:, q * H:(q + 1) * H]


def _mlp_l_body(eps_ref, x_ref, h_ref, w1a_ref, w1b_ref, b1_ref, w2_ref,
                b2_ref, g_ref, beta_ref, o_ref):
    eps1 = eps_ref[0]
    pre = eps1 * _cat(x_ref) + _cat(h_ref)
    up = pltpu.roll(pre, _RB - 1, 0)
    dn = pltpu.roll(pre, 1, 0)
    rid = lax.broadcasted_iota(jnp.int32, pre.shape, 0)
    sw = jnp.where((rid % 2) == 0, up, dn)
    a = (jnp.dot(pre, w1a_ref[...], preferred_element_type=jnp.float32)
         + jnp.dot(sw, w1b_ref[...], preferred_element_type=jnp.float32)
         + b1_ref[...])
    a = jnp.maximum(a, 0.0)
    hb = jnp.dot(a, w2_ref[...], preferred_element_type=jnp.float32) + b2_ref[...]
    y = _layernorm(hb, g_ref[...], beta_ref[...])
    for q in range(NQ):
        o_ref[q] = y[:, q * H:(q + 1) * H]


_RB = 1000


def _row_spec():
    return pl.BlockSpec((NQ, _RB, H), lambda i: (0, i, 0))


def _full_spec():
    return pl.BlockSpec((D, D), lambda i: (0, 0))


def _vec_spec():
    return pl.BlockSpec((1, D), lambda i: (0, 0))


def _mlp_c(eps1, x_q, h_q, w1, b1, w2, b2, g, beta):
    return pl.pallas_call(
        _mlp_c_body,
        grid=(NC // _RB,),
        in_specs=[
            pl.BlockSpec(memory_space=pltpu.SMEM),
            _row_spec(), _row_spec(),
            _full_spec(), _vec_spec(), _full_spec(), _vec_spec(),
            _vec_spec(), _vec_spec(),
        ],
        out_specs=_row_spec(),
        out_shape=jax.ShapeDtypeStruct((NQ, NC, H), jnp.float32),
    )(eps1, x_q, h_q, w1, b1, w2, b2, g, beta)


def _mlp_l(eps1, x_q, h_q, w1a, w1b, b1, w2, b2, g, beta):
    return pl.pallas_call(
        _mlp_l_body,
        grid=(NL // _RB,),
        in_specs=[
            pl.BlockSpec(memory_space=pltpu.SMEM),
            _row_spec(), _row_spec(),
            _full_spec(), _full_spec(), _vec_spec(), _full_spec(),
            _vec_spec(), _vec_spec(), _vec_spec(),
        ],
        out_specs=_row_spec(),
        out_shape=jax.ShapeDtypeStruct((NQ, NL, H), jnp.float32),
    )(eps1, x_q, h_q, w1a, w1b, b1, w2, b2, g, beta)


def kernel(edge_index, L_init, C_init, epsilon, L_W1, L_b1, L_W2, L_b2,
           L_g, L_beta, C_W1, C_b1, C_W2, C_b2, C_g, C_beta):
    f32 = jnp.float32
    src = edge_index[0].astype(jnp.int32)
    dst = edge_index[1].astype(jnp.int32)
    npad = E_PAD - E
    gpad = jnp.zeros((npad,), jnp.int32)
    spad = NL + (jnp.arange(npad, dtype=jnp.int32) % NTILES)

    def mk_gidx(idx):
        a = jnp.concatenate([idx, gpad]).reshape(NTILES, NCHUNK, CHUNK)
        return jnp.stack([a + (q % 2) * NL for q in range(NQ)])

    def mk_sidx(idx):
        return jnp.concatenate([idx, spad]).reshape(NTILES, NCHUNK, CHUNK)

    g_l2c, s_l2c = mk_gidx(src), mk_sidx(dst)
    g_c2l, s_c2l = mk_gidx(dst), mk_sidx(src)
    zeros_acc = jnp.zeros((ACC_ROWS, H), f32)
    eps1 = (epsilon + 1.0).astype(f32)

    def to_q(x):
        return jnp.stack([x[:, q * H:(q + 1) * H] for q in range(NQ)])

    scale = np.float32(1.0 / np.sqrt(D))
    lits_q = to_q(jnp.broadcast_to(L_init * scale, (NL, D)))
    cls_q = to_q(jnp.broadcast_to(C_init * scale, (NC, D)))

    for i in range(ITERS):
        h_c = _segsum(lits_q.reshape(NQ * NL // 2, 2 * H), g_l2c, s_l2c,
                      zeros_acc)
        cls_q = _mlp_c(eps1, cls_q, h_c, C_W1[i], C_b1[i].reshape(1, D),
                       C_W2[i], C_b2[i].reshape(1, D), C_g[i].reshape(1, D),
                       C_beta[i].reshape(1, D))
        h_l = _segsum(cls_q.reshape(NQ * NC // 2, 2 * H), g_c2l, s_c2l,
                      zeros_acc)
        lits_q = _mlp_l(eps1, lits_q, h_l, L_W1[i, :D], L_W1[i, D:],
                        L_b1[i].reshape(1, D), L_W2[i],
                        L_b2[i].reshape(1, D), L_g[i].reshape(1, D),
                        L_beta[i].reshape(1, D))

    lits_out = jnp.concatenate([lits_q[q] for q in range(NQ)], axis=-1)
    cls_out = jnp.concatenate([cls_q[q] for q in range(NQ)], axis=-1)
    return (lits_out, cls_out)


# speedup vs baseline: 2.5201x; 2.5201x over previous
import functools

import numpy as np
import jax
import jax.numpy as jnp
from jax import lax
from jax.experimental import pallas as pl
from jax.experimental.pallas import tpu as pltpu
from jax.experimental.pallas import tpu_sc as plsc

NL = 10000
NC = 10000
E = 160000
D = 256
NQ = 4
H = D // NQ
ITERS = 3

NCORES = 2
NPASS = NQ // NCORES
NTILES = 16
CHUNK = 128
EP_TILE = 10240
NCHUNK = EP_TILE // CHUNK
E_PAD = EP_TILE * NTILES
NBUF = 4

N_PAD = 10112
TRPT = N_PAD // NTILES
DST_H = N_PAD // 2
GARB = 576
ACC_ROWS = DST_H + GARB
RPT = ACC_ROWS // NTILES
WRT = DST_H // 8
PADV = 1 << 20


def _segsum_body(x_hbm, gidx_hbm, sidx_hbm, zeros_hbm, out_hbm,
                 gidx_v, sidx_v, rows_v, table_s, acc_s,
                 g0, g1, g2, g3, s0, s1, s2, s3):
    c = lax.axis_index("c")
    s = lax.axis_index("s")
    gsems = (g0, g1, g2, g3)
    ssems = (s0, s1, s2, s3)
    pltpu.sync_copy(gidx_hbm.at[s], gidx_v)
    pltpu.sync_copy(sidx_hbm.at[0, s], sidx_v.at[0])
    pltpu.sync_copy(sidx_hbm.at[1, s], sidx_v.at[1])

    for p in range(NPASS):
        q = c * NPASS + p
        pltpu.sync_copy(x_hbm.at[pl.ds(q * N_PAD + s * TRPT, TRPT)],
                        table_s.at[pl.ds(s * TRPT, TRPT)])
        for h in range(2):
            pltpu.sync_copy(zeros_hbm.at[pl.ds(s * RPT, RPT)],
                            acc_s.at[pl.ds(s * RPT, RPT)])
            plsc.subcore_barrier()

            for b in range(NBUF):
                pltpu.async_copy(table_s.at[gidx_v.at[b]], rows_v.at[b],
                                 gsems[b])

            def outer(o, carry):
                for b in range(NBUF):
                    j = o * NBUF + b
                    pltpu.make_async_copy(table_s.at[gidx_v.at[j]],
                                          rows_v.at[b], gsems[b]).wait()
                    pltpu.async_copy(rows_v.at[b], acc_s.at[sidx_v.at[h, j]],
                                     ssems[b], add=True)
                    bp = (b - 1) % NBUF
                    jp = j - 1 + NBUF

                    @pl.when(jnp.logical_and(jp >= NBUF, jp < NCHUNK))
                    def _():
                        pltpu.make_async_copy(rows_v.at[bp],
                                              acc_s.at[sidx_v.at[h, j - 1]],
                                              ssems[bp]).wait()
                        pltpu.async_copy(table_s.at[gidx_v.at[jp]],
                                         rows_v.at[bp], gsems[bp])
                return carry

            lax.fori_loop(0, NCHUNK // NBUF, outer, 0)
            for b in range(NBUF):
                jlast = NCHUNK - NBUF + b
                pltpu.make_async_copy(rows_v.at[b],
                                      acc_s.at[sidx_v.at[h, jlast]],
                                      ssems[b]).wait()

            plsc.subcore_barrier()
            @pl.when(s < 8)
            def _():
                pltpu.sync_copy(
                    acc_s.at[pl.ds(s * WRT, WRT)],
                    out_hbm.at[q, pl.ds(h * DST_H + s * WRT, WRT)])
            plsc.subcore_barrier()


@functools.cache
def _make_segsum():
    return pl.kernel(
        _segsum_body,
        out_type=jax.ShapeDtypeStruct((NQ, N_PAD, H), jnp.float32),
        mesh=plsc.VectorSubcoreMesh(core_axis_name="c", subcore_axis_name="s",
                                    num_cores=NCORES, num_subcores=NTILES),
        scratch_types=[
            pltpu.VMEM((NCHUNK, CHUNK), jnp.int32),
            pltpu.VMEM((2, NCHUNK, CHUNK), jnp.int32),
            pltpu.VMEM((NBUF, CHUNK, H), jnp.float32),
            pltpu.VMEM_SHARED((N_PAD, H), jnp.float32),
            pltpu.VMEM_SHARED((ACC_ROWS, H), jnp.float32),
        ] + [pltpu.SemaphoreType.DMA] * (2 * NBUF),
        compiler_params=pltpu.CompilerParams(use_tc_tiling_on_sc=False),
    )


def _segsum(*args):
    return _make_segsum()(*args)


def _layernorm(h, g, beta):
    mu = jnp.mean(h, axis=-1, keepdims=True)
    var = jnp.mean((h - mu) * (h - mu), axis=-1, keepdims=True)
    return (h - mu) * lax.rsqrt(var + 1e-5) * g + beta


def _cat(ref):
    return jnp.concatenate([ref[q] for q in range(NQ)], axis=-1)


def _mlp_c_body(eps_ref, x_ref, h_ref, w1_ref, b1_ref, w2_ref, b2_ref,
                g_ref, beta_ref, o_ref):
    eps1 = eps_ref[0]
    pre = eps1 * _cat(x_ref) + _cat(h_ref)
    a = jnp.dot(pre, w1_ref[...], preferred_element_type=jnp.float32) + b1_ref[...]
    a = jnp.maximum(a, 0.0)
    hb = jnp.dot(a, w2_ref[...], preferred_element_type=jnp.float32) + b2_ref[...]
    y = _layernorm(hb, g_ref[...], beta_ref[...])
    for q in range(NQ):
        o_ref[q] = y[:, q * H:(q + 1) * H]


def _mlp_l_body(eps_ref, x_ref, h_ref, w1a_ref, w1b_ref, b1_ref, w2_ref,
                b2_ref, g_ref, beta_ref, o_ref):
    eps1 = eps_ref[0]
    pre = eps1 * _cat(x_ref) + _cat(h_ref)
    up = pltpu.roll(pre, _RB - 1, 0)
    dn = pltpu.roll(pre, 1, 0)
    rid = lax.broadcasted_iota(jnp.int32, pre.shape, 0)
    sw = jnp.where((rid % 2) == 0, up, dn)
    a = (jnp.dot(pre, w1a_ref[...], preferred_element_type=jnp.float32)
         + jnp.dot(sw, w1b_ref[...], preferred_element_type=jnp.float32)
         + b1_ref[...])
    a = jnp.maximum(a, 0.0)
    hb = jnp.dot(a, w2_ref[...], preferred_element_type=jnp.float32) + b2_ref[...]
    y = _layernorm(hb, g_ref[...], beta_ref[...])
    for q in range(NQ):
        o_ref[q] = y[:, q * H:(q + 1) * H]


_RB = 1000


def _row_spec():
    return pl.BlockSpec((NQ, _RB, H), lambda i: (0, i, 0))


def _full_spec():
    return pl.BlockSpec((D, D), lambda i: (0, 0))


def _vec_spec():
    return pl.BlockSpec((1, D), lambda i: (0, 0))


def _mlp_c(eps1, x_q, h_q, w1, b1, w2, b2, g, beta):
    return pl.pallas_call(
        _mlp_c_body,
        grid=(NC // _RB,),
        in_specs=[
            pl.BlockSpec(memory_space=pltpu.SMEM),
            _row_spec(), _row_spec(),
            _full_spec(), _vec_spec(), _full_spec(), _vec_spec(),
            _vec_spec(), _vec_spec(),
        ],
        out_specs=_row_spec(),
        out_shape=jax.ShapeDtypeStruct((NQ, N_PAD, H), jnp.float32),
    )(eps1, x_q, h_q, w1, b1, w2, b2, g, beta)


def _mlp_l(eps1, x_q, h_q, w1a, w1b, b1, w2, b2, g, beta):
    return pl.pallas_call(
        _mlp_l_body,
        grid=(NL // _RB,),
        in_specs=[
            pl.BlockSpec(memory_space=pltpu.SMEM),
            _row_spec(), _row_spec(),
            _full_spec(), _full_spec(), _vec_spec(), _full_spec(),
            _vec_spec(), _vec_spec(), _vec_spec(),
        ],
        out_specs=_row_spec(),
        out_shape=jax.ShapeDtypeStruct((NQ, N_PAD, H), jnp.float32),
    )(eps1, x_q, h_q, w1a, w1b, b1, w2, b2, g, beta)


def kernel(edge_index, L_init, C_init, epsilon, L_W1, L_b1, L_W2, L_b2,
           L_g, L_beta, C_W1, C_b1, C_W2, C_b2, C_g, C_beta):
    f32 = jnp.float32
    src = edge_index[0].astype(jnp.int32)
    dst = edge_index[1].astype(jnp.int32)
    npad = E_PAD - E
    gpad = jnp.zeros((npad,), jnp.int32)
    spad = jnp.full((npad,), PADV, jnp.int32)
    spread = jnp.arange(E_PAD, dtype=jnp.int32) % GARB

    def mk_gidx(idx):
        return jnp.concatenate([idx, gpad]).reshape(NTILES, NCHUNK, CHUNK)

    def mk_sidx(idx):
        a = jnp.concatenate([idx, spad])
        halves = []
        for h in range(2):
            local = a - h * DST_H
            ok = (local >= 0) & (local < DST_H)
            halves.append(jnp.where(ok, local, DST_H + spread))
        return jnp.stack(halves).reshape(2, NTILES, NCHUNK, CHUNK)

    g_l2c, s_l2c = mk_gidx(src), mk_sidx(dst)
    g_c2l, s_c2l = mk_gidx(dst), mk_sidx(src)
    zeros_acc = jnp.zeros((ACC_ROWS, H), f32)
    eps1 = (epsilon + 1.0).astype(f32)

    def to_q(x):
        q = jnp.stack([x[:, i * H:(i + 1) * H] for i in range(NQ)])
        return jnp.concatenate(
            [q, jnp.zeros((NQ, N_PAD - q.shape[1], H), f32)], axis=1)

    scale = np.float32(1.0 / np.sqrt(D))
    lits_q = to_q(jnp.broadcast_to(L_init * scale, (NL, D)))
    cls_q = to_q(jnp.broadcast_to(C_init * scale, (NC, D)))

    for i in range(ITERS):
        h_c = _segsum(lits_q.reshape(NQ * N_PAD, H), g_l2c, s_l2c, zeros_acc)
        cls_q = _mlp_c(eps1, cls_q, h_c, C_W1[i], C_b1[i].reshape(1, D),
                       C_W2[i], C_b2[i].reshape(1, D), C_g[i].reshape(1, D),
                       C_beta[i].reshape(1, D))
        h_l = _segsum(cls_q.reshape(NQ * N_PAD, H), g_c2l, s_c2l, zeros_acc)
        lits_q = _mlp_l(eps1, lits_q, h_l, L_W1[i, :D], L_W1[i, D:],
                        L_b1[i].reshape(1, D), L_W2[i],
                        L_b2[i].reshape(1, D), L_g[i].reshape(1, D),
                        L_beta[i].reshape(1, D))

    lits_out = jnp.concatenate([lits_q[q, :NL] for q in range(NQ)], axis=-1)
    cls_out = jnp.concatenate([cls_q[q, :NC] for q in range(NQ)], axis=-1)
    return (lits_out, cls_out)
